# Initial kernel scaffold; baseline (speedup 1.0000x reference)
import jax
import jax.numpy as jnp
from jax.experimental import pallas as pl


def kernel(x, edge_index, edge_attr, batch, W_in, b_in, W_e, b_e, eps, W1, b1, W2, b2, W_jk, b_jk, W_r, b_r, W_out, b_out):
    raise NotImplementedError("write your pallas kernel here")


# speedup vs baseline: 3.2437x; 3.2437x over previous
import functools

import jax
import jax.numpy as jnp
from jax import lax
from jax.experimental import pallas as pl
from jax.experimental.pallas import tpu as pltpu
from jax.experimental.pallas import tpu_sc as plsc

N = 10000
E = 320000
F = 128
ED = 16
H = 128
L = 5
G = 256

NC = 2
NS = 16
NW = NC * NS
C = 128
NCHUNKS = E // C
CH_PER_W = -(-NCHUNKS // NW)
ROWS_MAIN = 624
ROWS_TAIL_OFF = ROWS_MAIN * NS
ROWS_TAIL = N - ROWS_TAIL_OFF

VEC = 16


def _mm_relu_body(x_ref, w_ref, b_ref, o_ref):
    acc = jnp.dot(x_ref[...], w_ref[...], preferred_element_type=jnp.float32)
    o_ref[...] = jnp.maximum(acc + b_ref[...], 0.0)


def _encode_nodes(x, W_in, b_in):
    return pl.pallas_call(
        _mm_relu_body,
        grid=(5,),
        in_specs=[
            pl.BlockSpec((2000, F), lambda i: (i, 0)),
            pl.BlockSpec((F, H), lambda i: (0, 0)),
            pl.BlockSpec((1, H), lambda i: (0, 0)),
        ],
        out_specs=pl.BlockSpec((2000, H), lambda i: (i, 0)),
        out_shape=jax.ShapeDtypeStruct((N, H), jnp.float32),
    )(x, W_in, b_in.reshape(1, H))


def _encode_edges(edge_attr, W_e, b_e):
    return pl.pallas_call(
        _mm_relu_body,
        grid=(40,),
        in_specs=[
            pl.BlockSpec((8000, ED), lambda i: (i, 0)),
            pl.BlockSpec((ED, H), lambda i: (0, 0)),
            pl.BlockSpec((1, H), lambda i: (0, 0)),
        ],
        out_specs=pl.BlockSpec((8000, H), lambda i: (i, 0)),
        out_shape=jax.ShapeDtypeStruct((E, H), jnp.float32),
    )(edge_attr, W_e, b_e.reshape(1, H))


def _layer_mlp_body(eps_ref, h_ref, a0_ref, a1_ref, w1_ref, b1_ref, w2_ref,
                    b2_ref, o_ref):
    z = (1.0 + eps_ref[0]) * h_ref[...] + (a0_ref[...] + a1_ref[...])
    z = jnp.maximum(
        jnp.dot(z, w1_ref[...], preferred_element_type=jnp.float32)
        + b1_ref[...], 0.0)
    z = jnp.maximum(
        jnp.dot(z, w2_ref[...], preferred_element_type=jnp.float32)
        + b2_ref[...], 0.0)
    o_ref[...] = z


def _layer_mlp(eps_l, h, a0, a1, W1l, b1l, W2l, b2l):
    return pl.pallas_call(
        _layer_mlp_body,
        grid=(5,),
        in_specs=[
            pl.BlockSpec(memory_space=pltpu.SMEM),
            pl.BlockSpec((2000, H), lambda i: (i, 0)),
            pl.BlockSpec((2000, H), lambda i: (i, 0)),
            pl.BlockSpec((2000, H), lambda i: (i, 0)),
            pl.BlockSpec((H, H), lambda i: (0, 0)),
            pl.BlockSpec((1, H), lambda i: (0, 0)),
            pl.BlockSpec((H, H), lambda i: (0, 0)),
            pl.BlockSpec((1, H), lambda i: (0, 0)),
        ],
        out_specs=pl.BlockSpec((2000, H), lambda i: (i, 0)),
        out_shape=jax.ShapeDtypeStruct((N, H), jnp.float32),
    )(eps_l, h, a0, a1, W1l, b1l.reshape(1, H), W2l, b2l.reshape(1, H))


_FBLK = 2000


def _readout_body(batch_ref, h1, h2, h3, h4, h5, wjk, bjk, wr, br, wo, bo,
                  o_ref, s_sums, s_cnt):
    i = pl.program_id(0)

    @pl.when(i == 0)
    def _init():
        s_sums[...] = jnp.zeros_like(s_sums)
        s_cnt[...] = jnp.zeros_like(s_cnt)

    ids = batch_ref[0]
    mask_t = (lax.broadcasted_iota(jnp.int32, (G, _FBLK), 0) == ids
              ).astype(jnp.float32)
    for j, h_ref in enumerate((h1, h2, h3, h4, h5)):
        s_sums[:, j * H:(j + 1) * H] += jnp.dot(
            mask_t, h_ref[...], preferred_element_type=jnp.float32)
    s_cnt[...] += jnp.sum(mask_t, axis=1, keepdims=True)

    @pl.when(i == pl.num_programs(0) - 1)
    def _finish():
        graph = s_sums[...] / jnp.maximum(s_cnt[...], 1.0)
        hg = jnp.dot(graph, wjk[...], preferred_element_type=jnp.float32)
        hg = hg + bjk[...]
        r = jnp.maximum(
            jnp.dot(hg, wr[...], preferred_element_type=jnp.float32)
            + br[...], 0.0)
        logits = jnp.dot(r, wo[...], preferred_element_type=jnp.float32)
        o_ref[...] = jax.nn.sigmoid(logits + bo[...])


def _readout(batch, hs, W_jk, b_jk, W_r, b_r, W_out, b_out):
    batch3d = batch.reshape(N // _FBLK, 1, _FBLK)
    h_specs = [pl.BlockSpec((_FBLK, H), lambda i: (i, 0)) for _ in range(L)]
    w_specs = [
        pl.BlockSpec((L * H, H), lambda i: (0, 0)),
        pl.BlockSpec((1, H), lambda i: (0, 0)),
        pl.BlockSpec((H, H), lambda i: (0, 0)),
        pl.BlockSpec((1, H), lambda i: (0, 0)),
        pl.BlockSpec((H, 1), lambda i: (0, 0)),
        pl.BlockSpec((1, 1), lambda i: (0, 0)),
    ]
    return pl.pallas_call(
        _readout_body,
        grid=(N // _FBLK,),
        in_specs=[pl.BlockSpec((1, 1, _FBLK), lambda i: (i, 0, 0))]
        + h_specs + w_specs,
        out_specs=pl.BlockSpec((G, 1), lambda i: (0, 0)),
        out_shape=jax.ShapeDtypeStruct((G, 1), jnp.float32),
        scratch_shapes=[
            pltpu.VMEM((G, L * H), jnp.float32),
            pltpu.VMEM((G, 1), jnp.float32),
        ],
    )(batch3d, *hs, W_jk, b_jk.reshape(1, H), W_r, b_r.reshape(1, H),
      W_out, b_out.reshape(1, 1))


def _sc_aggregate(h, e, src, dst, zeros):
    mesh = plsc.VectorSubcoreMesh(core_axis_name="c", subcore_axis_name="s")

    @functools.partial(
        pl.kernel,
        out_type=jax.ShapeDtypeStruct((NC * N, H), jnp.float32),
        mesh=mesh,
        scratch_types=[
            pltpu.VMEM((C,), jnp.int32),
            pltpu.VMEM((C,), jnp.int32),
            pltpu.VMEM((C, H), jnp.float32),
            pltpu.VMEM((C, H), jnp.float32),
            pltpu.VMEM_SHARED((N, H), jnp.float32),
            pltpu.SemaphoreType.DMA,
        ],
    )
    def body(h_hbm, e_hbm, src_hbm, dst_hbm, z_hbm, out_hbm,
             src_v, dst_v, rows_v, e_v, acc_sh, sem):
        cid = lax.axis_index("c")
        sid = lax.axis_index("s")
        wid = sid * NC + cid

        r0 = pl.multiple_of(sid * ROWS_MAIN, 8)
        pltpu.sync_copy(z_hbm.at[pl.ds(r0, ROWS_MAIN)],
                        acc_sh.at[pl.ds(r0, ROWS_MAIN)])

        @pl.when(sid == NS - 1)
        def _zero_tail():
            pltpu.sync_copy(z_hbm.at[pl.ds(ROWS_TAIL_OFF, ROWS_TAIL)],
                            acc_sh.at[pl.ds(ROWS_TAIL_OFF, ROWS_TAIL)])

        plsc.subcore_barrier()

        def chunk_body(i, carry):
            chunk = i * NW + wid

            @pl.when(chunk < NCHUNKS)
            def _():
                base = chunk * C
                pltpu.sync_copy(src_hbm.at[pl.ds(base, C)], src_v)
                pltpu.sync_copy(dst_hbm.at[pl.ds(base, C)], dst_v)
                pltpu.async_copy(h_hbm.at[src_v], rows_v, sem).wait()
                pltpu.sync_copy(e_hbm.at[pl.ds(base, C)], e_v)

                def row_body(r, rc):
                    for j in range(H // VEC):
                        sl = pl.ds(j * VEC, VEC)
                        rows_v[r, sl] = jnp.maximum(
                            rows_v[r, sl] + e_v[r, sl], 0.0)
                    return rc

                lax.fori_loop(0, C, row_body, 0)
                pltpu.sync_copy(rows_v, acc_sh.at[dst_v], add=True)

            return carry

        lax.fori_loop(0, CH_PER_W, chunk_body, 0)
        plsc.subcore_barrier()

        o0 = pl.multiple_of(cid * N + r0, 8)
        pltpu.sync_copy(acc_sh.at[pl.ds(r0, ROWS_MAIN)],
                        out_hbm.at[pl.ds(o0, ROWS_MAIN)])

        @pl.when(sid == NS - 1)
        def _pub_tail():
            ot = pl.multiple_of(cid * N + ROWS_TAIL_OFF, 8)
            pltpu.sync_copy(acc_sh.at[pl.ds(ROWS_TAIL_OFF, ROWS_TAIL)],
                            out_hbm.at[pl.ds(ot, ROWS_TAIL)])

    return body(h, e, src, dst, zeros)


def kernel(x, edge_index, edge_attr, batch, W_in, b_in, W_e, b_e, eps,
           W1, b1, W2, b2, W_jk, b_jk, W_r, b_r, W_out, b_out):
    src = edge_index[0]
    dst = edge_index[1]
    zeros = jnp.zeros((N, H), jnp.float32)

    h = _encode_nodes(x, W_in, b_in)
    e = _encode_edges(edge_attr, W_e, b_e)

    hs = []
    for l in range(L):
        parts = _sc_aggregate(h, e, src, dst, zeros)
        a0 = parts[:N]
        a1 = parts[N:]
        h = _layer_mlp(eps[l].reshape(1), h, a0, a1, W1[l], b1[l], W2[l],
                       b2[l])
        hs.append(h)

    return _readout(batch, hs, W_jk, b_jk, W_r, b_r, W_out, b_out)


# speedup vs baseline: 4.1451x; 1.2779x over previous
import functools

import jax
import jax.numpy as jnp
from jax import lax
from jax.experimental import pallas as pl
from jax.experimental.pallas import tpu as pltpu
from jax.experimental.pallas import tpu_sc as plsc

N = 10000
E = 320000
F = 128
ED = 16
H = 128
L = 5
G = 256

NC = 2
NS = 16
NW = NC * NS
C = 40
NCHUNKS = E // C
CH_PER_W = NCHUNKS // NW
ROWS_MAIN = 624
ROWS_TAIL_OFF = ROWS_MAIN * NS
ROWS_TAIL = N - ROWS_TAIL_OFF

VEC = 16


def _mm_relu_body(x_ref, w_ref, b_ref, o_ref):
    acc = jnp.dot(x_ref[...], w_ref[...], preferred_element_type=jnp.float32)
    o_ref[...] = jnp.maximum(acc + b_ref[...], 0.0)


def _encode_nodes(x, W_in, b_in):
    return pl.pallas_call(
        _mm_relu_body,
        grid=(5,),
        in_specs=[
            pl.BlockSpec((2000, F), lambda i: (i, 0)),
            pl.BlockSpec((F, H), lambda i: (0, 0)),
            pl.BlockSpec((1, H), lambda i: (0, 0)),
        ],
        out_specs=pl.BlockSpec((2000, H), lambda i: (i, 0)),
        out_shape=jax.ShapeDtypeStruct((N, H), jnp.float32),
    )(x, W_in, b_in.reshape(1, H))


def _encode_edges(edge_attr, W_e, b_e):
    return pl.pallas_call(
        _mm_relu_body,
        grid=(40,),
        in_specs=[
            pl.BlockSpec((8000, ED), lambda i: (i, 0)),
            pl.BlockSpec((ED, H), lambda i: (0, 0)),
            pl.BlockSpec((1, H), lambda i: (0, 0)),
        ],
        out_specs=pl.BlockSpec((8000, H), lambda i: (i, 0)),
        out_shape=jax.ShapeDtypeStruct((E, H), jnp.float32),
    )(edge_attr, W_e, b_e.reshape(1, H))


def _layer_mlp_body(eps_ref, h_ref, a0_ref, a1_ref, w1_ref, b1_ref, w2_ref,
                    b2_ref, o_ref):
    z = (1.0 + eps_ref[0]) * h_ref[...] + (a0_ref[...] + a1_ref[...])
    z = jnp.maximum(
        jnp.dot(z, w1_ref[...], preferred_element_type=jnp.float32)
        + b1_ref[...], 0.0)
    z = jnp.maximum(
        jnp.dot(z, w2_ref[...], preferred_element_type=jnp.float32)
        + b2_ref[...], 0.0)
    o_ref[...] = z


def _layer_mlp(eps_l, h, a0, a1, W1l, b1l, W2l, b2l):
    return pl.pallas_call(
        _layer_mlp_body,
        grid=(5,),
        in_specs=[
            pl.BlockSpec(memory_space=pltpu.SMEM),
            pl.BlockSpec((2000, H), lambda i: (i, 0)),
            pl.BlockSpec((2000, H), lambda i: (i, 0)),
            pl.BlockSpec((2000, H), lambda i: (i, 0)),
            pl.BlockSpec((H, H), lambda i: (0, 0)),
            pl.BlockSpec((1, H), lambda i: (0, 0)),
            pl.BlockSpec((H, H), lambda i: (0, 0)),
            pl.BlockSpec((1, H), lambda i: (0, 0)),
        ],
        out_specs=pl.BlockSpec((2000, H), lambda i: (i, 0)),
        out_shape=jax.ShapeDtypeStruct((N, H), jnp.float32),
    )(eps_l, h, a0, a1, W1l, b1l.reshape(1, H), W2l, b2l.reshape(1, H))


_FBLK = 2000


def _readout_body(batch_ref, h1, h2, h3, h4, h5, wjk, bjk, wr, br, wo, bo,
                  o_ref, s_sums, s_cnt):
    i = pl.program_id(0)

    @pl.when(i == 0)
    def _init():
        s_sums[...] = jnp.zeros_like(s_sums)
        s_cnt[...] = jnp.zeros_like(s_cnt)

    ids = batch_ref[0]
    mask_t = (lax.broadcasted_iota(jnp.int32, (G, _FBLK), 0) == ids
              ).astype(jnp.float32)
    for j, h_ref in enumerate((h1, h2, h3, h4, h5)):
        s_sums[:, j * H:(j + 1) * H] += jnp.dot(
            mask_t, h_ref[...], preferred_element_type=jnp.float32)
    s_cnt[...] += jnp.sum(mask_t, axis=1, keepdims=True)

    @pl.when(i == pl.num_programs(0) - 1)
    def _finish():
        graph = s_sums[...] / jnp.maximum(s_cnt[...], 1.0)
        hg = jnp.dot(graph, wjk[...], preferred_element_type=jnp.float32)
        hg = hg + bjk[...]
        r = jnp.maximum(
            jnp.dot(hg, wr[...], preferred_element_type=jnp.float32)
            + br[...], 0.0)
        logits = jnp.dot(r, wo[...], preferred_element_type=jnp.float32)
        o_ref[...] = jax.nn.sigmoid(logits + bo[...])


def _readout(batch, hs, W_jk, b_jk, W_r, b_r, W_out, b_out):
    batch3d = batch.reshape(N // _FBLK, 1, _FBLK)
    h_specs = [pl.BlockSpec((_FBLK, H), lambda i: (i, 0)) for _ in range(L)]
    w_specs = [
        pl.BlockSpec((L * H, H), lambda i: (0, 0)),
        pl.BlockSpec((1, H), lambda i: (0, 0)),
        pl.BlockSpec((H, H), lambda i: (0, 0)),
        pl.BlockSpec((1, H), lambda i: (0, 0)),
        pl.BlockSpec((H, 1), lambda i: (0, 0)),
        pl.BlockSpec((1, 1), lambda i: (0, 0)),
    ]
    return pl.pallas_call(
        _readout_body,
        grid=(N // _FBLK,),
        in_specs=[pl.BlockSpec((1, 1, _FBLK), lambda i: (i, 0, 0))]
        + h_specs + w_specs,
        out_specs=pl.BlockSpec((G, 1), lambda i: (0, 0)),
        out_shape=jax.ShapeDtypeStruct((G, 1), jnp.float32),
        scratch_shapes=[
            pltpu.VMEM((G, L * H), jnp.float32),
            pltpu.VMEM((G, 1), jnp.float32),
        ],
    )(batch3d, *hs, W_jk, b_jk.reshape(1, H), W_r, b_r.reshape(1, H),
      W_out, b_out.reshape(1, 1))


_NSTAGE = 4
_NITER = 63


def _sc_aggregate(h, e, src, dst, zeros):
    mesh = plsc.VectorSubcoreMesh(core_axis_name="c", subcore_axis_name="s")

    @functools.partial(
        pl.kernel,
        out_type=jax.ShapeDtypeStruct((NC * N, H), jnp.float32),
        mesh=mesh,
        scratch_types=[
            pltpu.VMEM((C,), jnp.int32),
            pltpu.VMEM((C,), jnp.int32),
            pltpu.VMEM((C,), jnp.int32),
            pltpu.VMEM((C,), jnp.int32),
            pltpu.VMEM((C,), jnp.int32),
            pltpu.VMEM((C,), jnp.int32),
            pltpu.VMEM((C, H), jnp.float32),
            pltpu.VMEM((C, H), jnp.float32),
            pltpu.VMEM((C, H), jnp.float32),
            pltpu.VMEM((C, H), jnp.float32),
            pltpu.VMEM((C, H), jnp.float32),
            pltpu.VMEM((C, H), jnp.float32),
            pltpu.VMEM_SHARED((N, H), jnp.float32),
            pltpu.SemaphoreType.DMA,
            pltpu.SemaphoreType.DMA,
            pltpu.SemaphoreType.DMA,
            pltpu.SemaphoreType.DMA,
            pltpu.SemaphoreType.DMA,
            pltpu.SemaphoreType.DMA,
        ],
    )
    def body(h_hbm, e_hbm, src_hbm, dst_hbm, z_hbm, out_hbm,
             src_v0, src_v1, dst_v0, dst_v1, dst_v2, dst_v3,
             rows_v0, rows_v1, e_v0, e_v1, msg_v0, msg_v1,
             acc_sh, sem_g0, sem_g1, sem_e0, sem_e1, sem_s0, sem_s1):
        cid = lax.axis_index("c")
        sid = lax.axis_index("s")
        wid = sid * NC + cid
        src_v = (src_v0, src_v1)
        dst_v = (dst_v0, dst_v1, dst_v2, dst_v3)
        rows_v = (rows_v0, rows_v1)
        e_v = (e_v0, e_v1)
        msg_v = (msg_v0, msg_v1)
        sem_g = (sem_g0, sem_g1)
        sem_e = (sem_e0, sem_e1)
        sem_s = (sem_s0, sem_s1)

        r0 = pl.multiple_of(sid * ROWS_MAIN, 8)
        pltpu.sync_copy(z_hbm.at[pl.ds(r0, ROWS_MAIN)],
                        acc_sh.at[pl.ds(r0, ROWS_MAIN)])

        @pl.when(sid == NS - 1)
        def _zero_tail():
            pltpu.sync_copy(z_hbm.at[pl.ds(ROWS_TAIL_OFF, ROWS_TAIL)],
                            acc_sh.at[pl.ds(ROWS_TAIL_OFF, ROWS_TAIL)])

        plsc.subcore_barrier()

        def valid(i):
            return (i * NW + wid) < NCHUNKS

        def prefetch(i, b, d):
            base = (i * NW + wid) * C
            pltpu.sync_copy(src_hbm.at[pl.ds(base, C)], src_v[b])
            pltpu.sync_copy(dst_hbm.at[pl.ds(base, C)], dst_v[d])
            pltpu.async_copy(h_hbm.at[src_v[b]], rows_v[b], sem_g[b])
            pltpu.async_copy(e_hbm.at[pl.ds(base, C)], e_v[b], sem_e[b])

        def stage(i, ofs):
            b = ofs % 2
            d = ofs % _NSTAGE

            @pl.when(valid(i))
            def _wait_inputs():
                pltpu.make_async_copy(
                    h_hbm.at[src_v[b]], rows_v[b], sem_g[b]).wait()
                pltpu.make_async_copy(
                    e_hbm.at[pl.ds(0, C)], e_v[b], sem_e[b]).wait()

            @pl.when(jnp.logical_and(i >= 2, valid(i - 2)))
            def _wait_prev_scatter():
                pltpu.make_async_copy(
                    msg_v[b], acc_sh.at[dst_v[(ofs + 2) % _NSTAGE]],
                    sem_s[b]).wait()

            @pl.when(valid(i))
            def _compute_and_issue():
                def row_body(r, rc):
                    for j in range(H // VEC):
                        sl = pl.ds(j * VEC, VEC)
                        msg_v[b][r, sl] = jnp.maximum(
                            rows_v[b][r, sl] + e_v[b][r, sl], 0.0)
                    return rc

                lax.fori_loop(0, C, row_body, 0)
                pltpu.async_copy(msg_v[b], acc_sh.at[dst_v[d]], sem_s[b],
                                 add=True)

                @pl.when(valid(i + 2))
                def _():
                    prefetch(i + 2, b, (ofs + 2) % _NSTAGE)

        prefetch(0, 0, 0)
        prefetch(1, 1, 1)

        def pipe_body(k, carry):
            for ofs in range(_NSTAGE):
                stage(k * _NSTAGE + ofs, ofs)
            return carry

        lax.fori_loop(0, _NITER, pipe_body, 0)

        last = _NSTAGE * _NITER - 2

        @pl.when(valid(last))
        def _drain():
            pltpu.make_async_copy(
                msg_v[0], acc_sh.at[dst_v[last % _NSTAGE]], sem_s[0]).wait()

        plsc.subcore_barrier()

        o0 = pl.multiple_of(cid * N + r0, 8)
        pltpu.sync_copy(acc_sh.at[pl.ds(r0, ROWS_MAIN)],
                        out_hbm.at[pl.ds(o0, ROWS_MAIN)])

        @pl.when(sid == NS - 1)
        def _pub_tail():
            ot = pl.multiple_of(cid * N + ROWS_TAIL_OFF, 8)
            pltpu.sync_copy(acc_sh.at[pl.ds(ROWS_TAIL_OFF, ROWS_TAIL)],
                            out_hbm.at[pl.ds(ot, ROWS_TAIL)])

    return body(h, e, src, dst, zeros)


def kernel(x, edge_index, edge_attr, batch, W_in, b_in, W_e, b_e, eps,
           W1, b1, W2, b2, W_jk, b_jk, W_r, b_r, W_out, b_out):
    src = edge_index[0]
    dst = edge_index[1]
    zeros = jnp.zeros((N, H), jnp.float32)

    h = _encode_nodes(x, W_in, b_in)
    e = _encode_edges(edge_attr, W_e, b_e)

    hs = []
    for l in range(L):
        parts = _sc_aggregate(h, e, src, dst, zeros)
        a0 = parts[:N]
        a1 = parts[N:]
        h = _layer_mlp(eps[l].reshape(1), h, a0, a1, W1[l], b1[l], W2[l],
                       b2[l])
        hs.append(h)

    return _readout(batch, hs, W_jk, b_jk, W_r, b_r, W_out, b_out)
